# Initial kernel scaffold; baseline (speedup 1.0000x reference)
import jax
import jax.numpy as jnp
from jax.experimental import pallas as pl


def kernel(Xi, Xv, emb1, emb2, W1, b1, g1, beta1, W2, b2, g2, beta2, bias):
    raise NotImplementedError("write your pallas kernel here")


# speedup vs baseline: 1.1477x; 1.1477x over previous
import functools

import jax
import jax.numpy as jnp
from jax import lax
from jax.experimental import pallas as pl
from jax.experimental.pallas import tpu as pltpu
from jax.experimental.pallas import tpu_sc as plsc

FIELDS = 26
VOCAB = 100000
BATCH = 16384
EMB = 16
D = FIELDS * EMB
H1 = 64
H2 = 32
EPS = 1e-5

NIDX = BATCH * FIELDS
NW = 32
PER_W = NIDX // NW
G = 128
NG_W = PER_W // G
CH_G = 8
CH = CH_G * G
NCH = PER_W // CH

def _sc_body(tab2, tab1, idx, deep_out, first_out,
             idx_v, rows_v, vals_v, acc_v, gsem, vsem):
    wid = lax.axis_index("s") * 2 + lax.axis_index("c")
    pltpu.sync_copy(idx.at[pl.ds(wid * NG_W, NG_W)], idx_v)

    def chunk(c, acc):
        rcopies = []
        vcopies = []
        for g in range(CH_G):
            row = c * CH_G + g
            rcopies.append(pltpu.make_async_copy(
                tab2.at[idx_v.at[row]], rows_v.at[pl.ds(g * G, G)], gsem))
            vcopies.append(pltpu.make_async_copy(
                tab1.at[idx_v.at[row]], vals_v.at[pl.ds(g * G, G)], vsem))
        for cp in rcopies:
            cp.start()
        for cp in vcopies:
            cp.start()
        for cp in rcopies:
            cp.wait()
        for cp in vcopies:
            cp.wait()
        pltpu.sync_copy(rows_v, deep_out.at[pl.ds(wid * PER_W + c * CH, CH)])

        def accum(j, a):
            return a + vals_v[pl.ds(j * 16, 16)]
        return lax.fori_loop(0, CH // 16, accum, acc)

    acc = lax.fori_loop(0, NCH, chunk, jnp.zeros((16,), jnp.float32))
    acc_v[...] = acc
    pltpu.sync_copy(acc_v, first_out.at[wid])


@functools.lru_cache(maxsize=None)
def _sc_gather():
    mesh = plsc.VectorSubcoreMesh(core_axis_name="c", subcore_axis_name="s")
    return pl.kernel(
        _sc_body,
        out_type=[
            jax.ShapeDtypeStruct((NIDX, EMB), jnp.float32),
            jax.ShapeDtypeStruct((NW, 16), jnp.float32),
        ],
        mesh=mesh,
        compiler_params=pltpu.CompilerParams(use_tc_tiling_on_sc=False),
        scratch_types=[
            pltpu.VMEM((NG_W, G), jnp.int32),
            pltpu.VMEM((CH, EMB), jnp.float32),
            pltpu.VMEM((CH,), jnp.float32),
            pltpu.VMEM((16,), jnp.float32),
            pltpu.SemaphoreType.DMA,
            pltpu.SemaphoreType.DMA,
        ],
    )


BLK = 2048


def _tc_body(deep_ref, xv_ref, w1_ref, b1_ref, g1_ref, be1_ref,
             w2_ref, b2_ref, g2_ref, be2_ref, fp_ref, bias_ref, out_ref):
    f32 = jnp.float32
    hi = lax.Precision.HIGHEST
    deep_raw = deep_ref[...]
    xv = xv_ref[...]
    ri = lax.broadcasted_iota(jnp.int32, (FIELDS, D), 0)
    ci = lax.broadcasted_iota(jnp.int32, (FIELDS, D), 1)
    expand = (ci // EMB == ri).astype(f32)
    scaled = deep_raw * jnp.dot(xv, expand, precision=hi,
                                preferred_element_type=f32)
    rj = lax.broadcasted_iota(jnp.int32, (D, EMB), 0)
    ck = lax.broadcasted_iota(jnp.int32, (D, EMB), 1)
    fsum = (rj % EMB == ck).astype(f32)
    fm_sum = jnp.dot(scaled, fsum, precision=hi, preferred_element_type=f32)
    fm2 = 0.5 * (jnp.sum(fm_sum * fm_sum, axis=1)
                 - jnp.sum(scaled * scaled, axis=1))
    inv = (1.0 + EPS) ** -0.5
    h = jnp.dot(scaled, w1_ref[...], precision=hi,
                preferred_element_type=f32) + b1_ref[...]
    h = jnp.maximum(h, 0.0) * (inv * g1_ref[...]) + be1_ref[...]
    h = jnp.dot(h, w2_ref[...], precision=hi,
                preferred_element_type=f32) + b2_ref[...]
    h = jnp.maximum(h, 0.0) * (inv * g2_ref[...]) + be2_ref[...]
    dsum = jnp.sum(h, axis=1)
    first = jnp.sum(fp_ref[...])
    out_ref[...] = fm2 + dsum + (first + bias_ref[0, 0])


def _tc_call(deep, xv, w1, b1, g1, be1, w2, b2, g2, be2, fparts, bias):
    full = lambda shape: pl.BlockSpec(shape, lambda i: (0,) * len(shape))
    return pl.pallas_call(
        _tc_body,
        grid=(BATCH // BLK,),
        in_specs=[
            pl.BlockSpec((BLK, D), lambda i: (i, 0)),
            pl.BlockSpec((BLK, FIELDS), lambda i: (i, 0)),
            full((D, H1)), full((1, H1)), full((1, H1)), full((1, H1)),
            full((H1, H2)), full((1, H2)), full((1, H2)), full((1, H2)),
            full((NW, 16)), full((1, 1)),
        ],
        out_specs=pl.BlockSpec((BLK,), lambda i: (i,)),
        out_shape=jax.ShapeDtypeStruct((BATCH,), jnp.float32),
    )(deep, xv, w1, b1, g1, be1, w2, b2, g2, be2, fparts, bias)


def kernel(Xi, Xv, emb1, emb2, W1, b1, g1, beta1, W2, b2, g2, beta2, bias):
    tab2 = emb2.reshape(FIELDS * VOCAB, EMB)
    tab1 = emb1.reshape(FIELDS * VOCAB)
    offs = (jnp.arange(FIELDS, dtype=jnp.int32) * VOCAB)[None, :]
    idx = (Xi[:, :, 0].astype(jnp.int32) + offs).reshape(NIDX // G, G)
    deep_raw, fparts = _sc_gather()(tab2, tab1, idx)
    return _tc_call(
        deep_raw.reshape(BATCH, D), Xv,
        W1, b1.reshape(1, H1), g1.reshape(1, H1), beta1.reshape(1, H1),
        W2, b2.reshape(1, H2), g2.reshape(1, H2), beta2.reshape(1, H2),
        fparts, bias.reshape(1, 1))
